# baseline (speedup 1.0000x reference)
import functools

import jax
import jax.numpy as jnp
from jax.experimental import pallas as pl
from jax.experimental.pallas import tpu as pltpu

B = 16384
N = 5
N_EDGES = 20
HID = 64
N_LAYERS = 4
G = 128


def _silu(x):
    return x * jax.nn.sigmoid(x)


def _dot(a, b):
    return jnp.dot(a, b, preferred_element_type=jnp.float32)


def _gnn_kernel(x_ref, ef0_ref, ef1_ref, ei_ref, *wrefs, out_ref):
    ws = [w[:] for w in wrefs]
    it = iter(ws)
    W_emb, b_emb = next(it), next(it)
    layers = [tuple(next(it) for _ in range(9)) for _ in range(N_LAYERS)]
    W_d1, b_d1, W_d2, b_d2 = next(it), next(it), next(it), next(it)

    ei = ei_ref[:]
    src = ei[:, :N_EDGES]
    tgt = ei[:, N_EDGES:]

    eidx = src * N + tgt
    ef0t = ef0_ref[:]
    ef1t = ef1_ref[:]
    c0 = jnp.zeros((G, N_EDGES), jnp.float32)
    c1 = jnp.zeros((G, N_EDGES), jnp.float32)
    for k in range(N * N):
        mk = (eidx == k).astype(jnp.float32)
        c0 = c0 + mk * ef0t[:, k:k + 1]
        c1 = c1 + mk * ef1t[:, k:k + 1]

    srcm = [(src == n).astype(jnp.float32) for n in range(N)]
    tgtm = [(tgt == n).astype(jnp.float32) for n in range(N)]

    x2d = _dot(x_ref[:], W_emb) + b_emb

    for li in range(N_LAYERS):
        Wcat, eW1e, eb1, eW2, eb2, nW1a, nb1, nW2, nb2 = layers[li]
        y = _dot(x2d, Wcat)
        y3 = y.reshape(G, N, 3 * HID)
        h3 = (c0[:, :, None] * eW1e[0:1, :][None, :, :]
              + c1[:, :, None] * eW1e[1:2, :][None, :, :])
        for n in range(N):
            h3 = h3 + srcm[n][:, :, None] * y3[:, n:n + 1, :HID]
            h3 = h3 + tgtm[n][:, :, None] * y3[:, n:n + 1, HID:2 * HID]
        h = h3.reshape(G * N_EDGES, HID) + eb1
        e = _silu(h)
        e = _silu(_dot(e, eW2) + eb2)
        e3 = e.reshape(G, N_EDGES, HID)
        aggs = [jnp.sum(srcm[n][:, :, None] * e3, axis=1, keepdims=True)
                for n in range(N)]
        agg2d = jnp.concatenate(aggs, axis=1).reshape(G * N, HID)
        hn = _silu(y[:, 2 * HID:] + _dot(agg2d, nW1a) + nb1)
        x2d = _dot(hn, nW2) + nb2

    d = _silu(_dot(x2d, W_d1) + b_d1)
    out_ref[:] = _dot(d, W_d2) + b_d2


def _body(x_ref, ef0_ref, ef1_ref, ei_ref, *rest):
    _gnn_kernel(x_ref, ef0_ref, ef1_ref, ei_ref, *rest[:-1], out_ref=rest[-1])


@jax.jit
def kernel(node_features, edge_features, edge_idx, params):
    b = node_features.shape[0]
    x_in = node_features.transpose(0, 1, 3, 2).reshape(b * N, 3 * 2)
    ef_flat = edge_features.reshape(b, N * N, 2)
    ef0_in = ef_flat[:, :, 0]
    ef1_in = ef_flat[:, :, 1]
    ei_in = edge_idx.astype(jnp.int32).reshape(b, 2 * N_EDGES)

    weights = [params['W_emb'], params['b_emb'].reshape(1, HID)]
    for i in range(N_LAYERS):
        p = params[f'layer_{i}']
        Wcat = jnp.concatenate(
            [p['eW1'][:HID], p['eW1'][HID:2 * HID], p['nW1'][:HID]], axis=1)
        weights += [
            Wcat, p['eW1'][2 * HID:],
            p['eb1'].reshape(1, HID),
            p['eW2'], p['eb2'].reshape(1, HID),
            p['nW1'][HID:],
            p['nb1'].reshape(1, HID),
            p['nW2'], p['nb2'].reshape(1, HID),
        ]
    weights += [params['W_d1'], params['b_d1'].reshape(1, HID),
                params['W_d2'], params['b_d2'].reshape(1, 3)]

    grid = (b // G,)
    data_specs = [
        pl.BlockSpec((G * N, 3 * 2), lambda i: (i, 0)),
        pl.BlockSpec((G, N * N), lambda i: (i, 0)),
        pl.BlockSpec((G, N * N), lambda i: (i, 0)),
        pl.BlockSpec((G, 2 * N_EDGES), lambda i: (i, 0)),
    ]
    w_specs = [pl.BlockSpec(w.shape, functools.partial(lambda nd, i: (0,) * nd,
                                                       w.ndim))
               for w in weights]
    out = pl.pallas_call(
        _body,
        grid=grid,
        in_specs=data_specs + w_specs,
        out_specs=pl.BlockSpec((G * N, 3), lambda i: (i, 0)),
        out_shape=jax.ShapeDtypeStruct((b * N, 3), jnp.float32),
        compiler_params=pltpu.CompilerParams(
            dimension_semantics=("parallel",)),
    )(x_in, ef0_in, ef1_in, ei_in, *weights)
    return out.reshape(b, N, 3)


# speedup vs baseline: 1.2969x; 1.2969x over previous
import functools

import jax
import jax.numpy as jnp
from jax.experimental import pallas as pl
from jax.experimental.pallas import tpu as pltpu

B = 16384
N = 5
N_EDGES = 20
HID = 64
N_LAYERS = 4
G = 128


def _silu(x):
    return x * jax.nn.sigmoid(x)


def _dot(a, b):
    return jnp.dot(a, b, preferred_element_type=jnp.float32)


def _gnn_kernel(x_ref, ef0_ref, ef1_ref, ei_ref, *wrefs, out_ref):
    ws = [w[:] for w in wrefs]
    it = iter(ws)
    W_emb, b_emb = next(it), next(it)
    layers = [tuple(next(it) for _ in range(11)) for _ in range(N_LAYERS)]
    W_d1, b_d1, W_d2, b_d2 = next(it), next(it), next(it), next(it)

    ei = ei_ref[:]
    src = ei[:, :N_EDGES]
    tgt = ei[:, N_EDGES:]

    eidx = src * N + tgt
    ef0t = ef0_ref[:]
    ef1t = ef1_ref[:]
    c0 = jnp.zeros((G, N_EDGES), jnp.float32)
    c1 = jnp.zeros((G, N_EDGES), jnp.float32)
    for k in range(N * N):
        mk = (eidx == k).astype(jnp.float32)
        c0 = c0 + mk * ef0t[:, k:k + 1]
        c1 = c1 + mk * ef1t[:, k:k + 1]

    srcm = [(src == n).astype(jnp.float32) for n in range(N)]
    tgtm = [(tgt == n).astype(jnp.float32) for n in range(N)]

    x2d = _dot(x_ref[:], W_emb) + b_emb

    for li in range(N_LAYERS):
        eW1s, eW1t, eW1e, eb1, eW2, eb2, nW1x, nW1a, nb1, nW2, nb2 = layers[li]
        ys3 = _dot(x2d, eW1s).reshape(G, N, HID)
        yt3 = _dot(x2d, eW1t).reshape(G, N, HID)
        yx = _dot(x2d, nW1x)
        hs = c0[:, :, None] * eW1e[0:1, :][None, :, :]
        ht = c1[:, :, None] * eW1e[1:2, :][None, :, :]
        for n in range(N):
            hs = hs + srcm[n][:, :, None] * ys3[:, n:n + 1, :]
            ht = ht + tgtm[n][:, :, None] * yt3[:, n:n + 1, :]
        h = (hs + ht).reshape(G * N_EDGES, HID) + eb1
        e = _silu(h)
        e = _silu(_dot(e, eW2) + eb2)
        e3 = e.reshape(G, N_EDGES, HID)
        aggs = [jnp.sum(srcm[n][:, :, None] * e3, axis=1, keepdims=True)
                for n in range(N)]
        agg2d = jnp.concatenate(aggs, axis=1).reshape(G * N, HID)
        hn = _silu(yx + _dot(agg2d, nW1a) + nb1)
        x2d = _dot(hn, nW2) + nb2

    d = _silu(_dot(x2d, W_d1) + b_d1)
    out_ref[:] = _dot(d, W_d2) + b_d2


def _body(x_ref, ef0_ref, ef1_ref, ei_ref, *rest):
    _gnn_kernel(x_ref, ef0_ref, ef1_ref, ei_ref, *rest[:-1], out_ref=rest[-1])


@jax.jit
def kernel(node_features, edge_features, edge_idx, params):
    b = node_features.shape[0]
    x_in = node_features.transpose(0, 1, 3, 2).reshape(b * N, 3 * 2)
    ef_flat = edge_features.reshape(b, N * N, 2)
    ef0_in = ef_flat[:, :, 0]
    ef1_in = ef_flat[:, :, 1]
    ei_in = edge_idx.astype(jnp.int32).reshape(b, 2 * N_EDGES)

    weights = [params['W_emb'], params['b_emb'].reshape(1, HID)]
    for i in range(N_LAYERS):
        p = params[f'layer_{i}']
        weights += [
            p['eW1'][:HID], p['eW1'][HID:2 * HID], p['eW1'][2 * HID:],
            p['eb1'].reshape(1, HID),
            p['eW2'], p['eb2'].reshape(1, HID),
            p['nW1'][:HID], p['nW1'][HID:],
            p['nb1'].reshape(1, HID),
            p['nW2'], p['nb2'].reshape(1, HID),
        ]
    weights += [params['W_d1'], params['b_d1'].reshape(1, HID),
                params['W_d2'], params['b_d2'].reshape(1, 3)]

    grid = (b // G,)
    data_specs = [
        pl.BlockSpec((G * N, 3 * 2), lambda i: (i, 0)),
        pl.BlockSpec((G, N * N), lambda i: (i, 0)),
        pl.BlockSpec((G, N * N), lambda i: (i, 0)),
        pl.BlockSpec((G, 2 * N_EDGES), lambda i: (i, 0)),
    ]
    w_specs = [pl.BlockSpec(w.shape, functools.partial(lambda nd, i: (0,) * nd,
                                                       w.ndim))
               for w in weights]
    out = pl.pallas_call(
        _body,
        grid=grid,
        in_specs=data_specs + w_specs,
        out_specs=pl.BlockSpec((G * N, 3), lambda i: (i, 0)),
        out_shape=jax.ShapeDtypeStruct((b * N, 3), jnp.float32),
    )(x_in, ef0_in, ef1_in, ei_in, *weights)
    return out.reshape(b, N, 3)


# speedup vs baseline: 2.0159x; 1.5544x over previous
import functools

import jax
import jax.numpy as jnp
from jax.experimental import pallas as pl

N = 5
N_EDGES = 20
HID = 64
N_LAYERS = 4
GH = 64


def _silu(x):
    return x * jax.nn.sigmoid(x)


def _dot(a, b):
    return jnp.dot(a, b, preferred_element_type=jnp.float32)


def _pair_w(W):
    z = jnp.zeros_like(W)
    return jnp.concatenate(
        [jnp.concatenate([W, z], axis=1), jnp.concatenate([z, W], axis=1)],
        axis=0)


def _pair_b(bias):
    return jnp.tile(bias.reshape(1, -1), (1, 2))


def _gnn_kernel(x_ref, ef_ref, ei_ref, *wrefs, out_ref):
    ws = [w[:] for w in wrefs]
    it = iter(ws)
    W_emb, b_emb = next(it), next(it)
    layers = [tuple(next(it) for _ in range(12)) for _ in range(N_LAYERS)]
    W_d1, b_d1, W_d2, b_d2 = next(it), next(it), next(it), next(it)

    ei = ei_ref[:]
    srcA, tgtA = ei[:, :20], ei[:, 20:40]
    srcB, tgtB = ei[:, 40:60], ei[:, 60:80]

    def _pack20(a, b_):
        a3 = jnp.broadcast_to(a[:, :, None], (GH, N_EDGES, HID))
        b3 = jnp.broadcast_to(b_[:, :, None], (GH, N_EDGES, HID))
        return jnp.concatenate([a3, b3], axis=2)

    srcp = _pack20(srcA, srcB)
    tgtp = _pack20(tgtA, tgtB)
    srcm = [(srcp == n).astype(jnp.float32) for n in range(N)]
    tgtm = [(tgtp == n).astype(jnp.float32) for n in range(N)]

    eft = ef_ref[:]
    eidxA = srcA * N + tgtA
    eidxB = srcB * N + tgtB
    z20 = jnp.zeros((GH, N_EDGES), jnp.float32)
    c0A, c1A, c0B, c1B = z20, z20, z20, z20
    for k in range(N * N):
        mA = (eidxA == k).astype(jnp.float32)
        mB = (eidxB == k).astype(jnp.float32)
        c0A = c0A + mA * eft[:, k:k + 1]
        c1A = c1A + mA * eft[:, 25 + k:26 + k]
        c0B = c0B + mB * eft[:, 50 + k:51 + k]
        c1B = c1B + mB * eft[:, 75 + k:76 + k]
    c0p = _pack20(c0A, c0B)
    c1p = _pack20(c1A, c1B)

    x2d = _dot(x_ref[:], W_emb) + b_emb

    for li in range(N_LAYERS):
        eW1s, eW1t, w0, w1, eb1, eW2, eb2, nW1x, nW1a, nb1, nW2, nb2 = \
            layers[li]
        ys3 = _dot(x2d, eW1s).reshape(GH, N, 2 * HID)
        yt3 = _dot(x2d, eW1t).reshape(GH, N, 2 * HID)
        yx = _dot(x2d, nW1x)
        hs = c0p * w0
        ht = c1p * w1
        for n in range(N):
            hs = hs + srcm[n] * ys3[:, n:n + 1, :]
            ht = ht + tgtm[n] * yt3[:, n:n + 1, :]
        h = (hs + ht).reshape(GH * N_EDGES, 2 * HID) + eb1
        e = _silu(h)
        e = _silu(_dot(e, eW2) + eb2)
        e3 = e.reshape(GH, N_EDGES, 2 * HID)
        aggs = [jnp.sum(srcm[n] * e3, axis=1, keepdims=True)
                for n in range(N)]
        agg2d = jnp.concatenate(aggs, axis=1).reshape(GH * N, 2 * HID)
        hn = _silu(yx + _dot(agg2d, nW1a) + nb1)
        x2d = _dot(hn, nW2) + nb2

    d = _silu(_dot(x2d, W_d1) + b_d1)
    out_ref[:] = _dot(d, W_d2) + b_d2


def _body(x_ref, ef_ref, ei_ref, *rest):
    _gnn_kernel(x_ref, ef_ref, ei_ref, *rest[:-1], out_ref=rest[-1])


@jax.jit
def kernel(node_features, edge_features, edge_idx, params):
    b = node_features.shape[0]
    bh = b // 2
    x_flat = node_features.transpose(0, 1, 3, 2).reshape(b * N, 3 * 2)
    x_in = jnp.concatenate([x_flat[:bh * N], x_flat[bh * N:]], axis=1)

    ef_flat = edge_features.reshape(b, N * N, 2)
    ef_half = jnp.concatenate([ef_flat[:, :, 0], ef_flat[:, :, 1]], axis=1)
    ef_in = jnp.concatenate([ef_half[:bh], ef_half[bh:]], axis=1)

    ei_flat = edge_idx.astype(jnp.int32).reshape(b, 2 * N_EDGES)
    ei_in = jnp.concatenate([ei_flat[:bh], ei_flat[bh:]], axis=1)

    weights = [_pair_w(params['W_emb']), _pair_b(params['b_emb'])]
    for i in range(N_LAYERS):
        p = params[f'layer_{i}']
        weights += [
            _pair_w(p['eW1'][:HID]), _pair_w(p['eW1'][HID:2 * HID]),
            jnp.tile(p['eW1'][2 * HID:2 * HID + 1], (1, 2)),
            jnp.tile(p['eW1'][2 * HID + 1:], (1, 2)),
            _pair_b(p['eb1']),
            _pair_w(p['eW2']), _pair_b(p['eb2']),
            _pair_w(p['nW1'][:HID]), _pair_w(p['nW1'][HID:]),
            _pair_b(p['nb1']),
            _pair_w(p['nW2']), _pair_b(p['nb2']),
        ]
    weights += [_pair_w(params['W_d1']), _pair_b(params['b_d1']),
                _pair_w(params['W_d2']), _pair_b(params['b_d2'])]

    grid = (bh // GH,)
    data_specs = [
        pl.BlockSpec((GH * N, 12), lambda i: (i, 0)),
        pl.BlockSpec((GH, 100), lambda i: (i, 0)),
        pl.BlockSpec((GH, 80), lambda i: (i, 0)),
    ]
    w_specs = [pl.BlockSpec(w.shape, functools.partial(lambda nd, i: (0,) * nd,
                                                       w.ndim))
               for w in weights]
    out = pl.pallas_call(
        _body,
        grid=grid,
        in_specs=data_specs + w_specs,
        out_specs=pl.BlockSpec((GH * N, 6), lambda i: (i, 0)),
        out_shape=jax.ShapeDtypeStruct((bh * N, 6), jnp.float32),
    )(x_in, ef_in, ei_in, *weights)
    out = jnp.concatenate([out[:, :3], out[:, 3:]], axis=0)
    return out.reshape(b, N, 3)


# speedup vs baseline: 2.8993x; 1.4382x over previous
import functools

import jax
import jax.numpy as jnp
from jax.experimental import pallas as pl

N = 5
N_EDGES = 20
HID = 64
N_LAYERS = 4
GH = 64


def _silu(x):
    return x * jax.nn.sigmoid(x)


def _dot(a, b):
    return jnp.dot(a, b, preferred_element_type=jnp.float32)


def _pair_w(W):
    z = jnp.zeros_like(W)
    return jnp.concatenate(
        [jnp.concatenate([W, z], axis=1), jnp.concatenate([z, W], axis=1)],
        axis=0)


def _pair_b(bias):
    return jnp.tile(bias.reshape(1, -1), (1, 2))


def _planes(valsA, valsB):
    return jnp.stack(
        [jnp.concatenate(
            [jnp.broadcast_to(valsA[:, e:e + 1], (GH, HID)),
             jnp.broadcast_to(valsB[:, e:e + 1], (GH, HID))], axis=1)
         for e in range(N_EDGES)], axis=0)


def _gnn_kernel(x_ref, ef_ref, ei_ref, *wrefs, out_ref):
    ws = [w[:] for w in wrefs]
    it = iter(ws)
    W_emb, b_emb = next(it), next(it)
    layers = [tuple(next(it) for _ in range(12)) for _ in range(N_LAYERS)]
    W_d1, b_d1, W_d2, b_d2 = next(it), next(it), next(it), next(it)

    ei = ei_ref[:]
    srcA, tgtA = ei[:, :20], ei[:, 20:40]
    srcB, tgtB = ei[:, 40:60], ei[:, 60:80]

    eft = ef_ref[:]
    eidxA = srcA * N + tgtA
    eidxB = srcB * N + tgtB
    z20 = jnp.zeros((GH, N_EDGES), jnp.float32)
    c0A, c1A, c0B, c1B = z20, z20, z20, z20
    for k in range(N * N):
        mA = (eidxA == k).astype(jnp.float32)
        mB = (eidxB == k).astype(jnp.float32)
        c0A = c0A + mA * eft[:, k:k + 1]
        c1A = c1A + mA * eft[:, 25 + k:26 + k]
        c0B = c0B + mB * eft[:, 50 + k:51 + k]
        c1B = c1B + mB * eft[:, 75 + k:76 + k]
    cp0 = _planes(c0A, c0B)
    cp1 = _planes(c1A, c1B)

    srcp = _planes(srcA.astype(jnp.float32), srcB.astype(jnp.float32))
    tgtp = _planes(tgtA.astype(jnp.float32), tgtB.astype(jnp.float32))
    srcm = [(srcp == n).astype(jnp.float32) for n in range(N)]
    tgtm = [(tgtp == n).astype(jnp.float32) for n in range(N)]

    x2d = _dot(x_ref[:].reshape(N * GH, 12), W_emb) + b_emb

    for li in range(N_LAYERS):
        eW1s, eW1t, w0, w1, eb1, eW2, eb2, nW1x, nW1a, nb1, nW2, nb2 = \
            layers[li]
        ys = _dot(x2d, eW1s).reshape(N, GH, 2 * HID)
        yt = _dot(x2d, eW1t).reshape(N, GH, 2 * HID)
        yx = _dot(x2d, nW1x)
        hs = cp0 * w0
        ht = cp1 * w1
        for n in range(N):
            hs = hs + srcm[n] * ys[n][None, :, :]
            ht = ht + tgtm[n] * yt[n][None, :, :]
        h = (hs + ht).reshape(N_EDGES * GH, 2 * HID) + eb1
        e = _silu(h)
        e = _silu(_dot(e, eW2) + eb2)
        e3 = e.reshape(N_EDGES, GH, 2 * HID)
        aggs = [jnp.sum(srcm[n] * e3, axis=0) for n in range(N)]
        agg2d = jnp.stack(aggs, axis=0).reshape(N * GH, 2 * HID)
        hn = _silu(yx + _dot(agg2d, nW1a) + nb1)
        x2d = _dot(hn, nW2) + nb2

    d = _silu(_dot(x2d, W_d1) + b_d1)
    out = _dot(d, W_d2) + b_d2
    out_ref[:] = out.reshape(N, GH, 6)


def _body(x_ref, ef_ref, ei_ref, *rest):
    _gnn_kernel(x_ref, ef_ref, ei_ref, *rest[:-1], out_ref=rest[-1])


@jax.jit
def kernel(node_features, edge_features, edge_idx, params):
    b = node_features.shape[0]
    bh = b // 2
    x_flat = node_features.transpose(0, 1, 3, 2).reshape(b, N, 3 * 2)
    x_pack = jnp.concatenate([x_flat[:bh], x_flat[bh:]], axis=2)
    x_in = x_pack.transpose(1, 0, 2)

    ef_flat = edge_features.reshape(b, N * N, 2)
    ef_half = jnp.concatenate([ef_flat[:, :, 0], ef_flat[:, :, 1]], axis=1)
    ef_in = jnp.concatenate([ef_half[:bh], ef_half[bh:]], axis=1)

    ei_flat = edge_idx.astype(jnp.int32).reshape(b, 2 * N_EDGES)
    ei_in = jnp.concatenate([ei_flat[:bh], ei_flat[bh:]], axis=1)

    weights = [_pair_w(params['W_emb']), _pair_b(params['b_emb'])]
    for i in range(N_LAYERS):
        p = params[f'layer_{i}']
        weights += [
            _pair_w(p['eW1'][:HID]), _pair_w(p['eW1'][HID:2 * HID]),
            jnp.tile(p['eW1'][2 * HID:2 * HID + 1], (1, 2)),
            jnp.tile(p['eW1'][2 * HID + 1:], (1, 2)),
            _pair_b(p['eb1']),
            _pair_w(p['eW2']), _pair_b(p['eb2']),
            _pair_w(p['nW1'][:HID]), _pair_w(p['nW1'][HID:]),
            _pair_b(p['nb1']),
            _pair_w(p['nW2']), _pair_b(p['nb2']),
        ]
    weights += [_pair_w(params['W_d1']), _pair_b(params['b_d1']),
                _pair_w(params['W_d2']), _pair_b(params['b_d2'])]

    grid = (bh // GH,)
    data_specs = [
        pl.BlockSpec((N, GH, 12), lambda i: (0, i, 0)),
        pl.BlockSpec((GH, 100), lambda i: (i, 0)),
        pl.BlockSpec((GH, 80), lambda i: (i, 0)),
    ]
    w_specs = [pl.BlockSpec(w.shape, functools.partial(lambda nd, i: (0,) * nd,
                                                       w.ndim))
               for w in weights]
    out = pl.pallas_call(
        _body,
        grid=grid,
        in_specs=data_specs + w_specs,
        out_specs=pl.BlockSpec((N, GH, 6), lambda i: (0, i, 0)),
        out_shape=jax.ShapeDtypeStruct((N, bh, 6), jnp.float32),
    )(x_in, ef_in, ei_in, *weights)
    out = out.transpose(1, 0, 2)
    out = jnp.concatenate([out[:, :, :3], out[:, :, 3:]], axis=0)
    return out


# speedup vs baseline: 3.3782x; 1.1652x over previous
import functools

import jax
import jax.numpy as jnp
from jax.experimental import pallas as pl

N = 5
N_EDGES = 20
HID = 64
N_LAYERS = 4
GH = 128


def _silu(x):
    return x * jax.nn.sigmoid(x)


def _dot(a, b):
    return jnp.dot(a, b, preferred_element_type=jnp.float32)


def _pair_w(W):
    z = jnp.zeros_like(W)
    return jnp.concatenate(
        [jnp.concatenate([W, z], axis=1), jnp.concatenate([z, W], axis=1)],
        axis=0)


def _pair_b(bias):
    return jnp.tile(bias.reshape(1, -1), (1, 2))


def _planes(valsA, valsB):
    return jnp.stack(
        [jnp.concatenate(
            [jnp.broadcast_to(valsA[:, e:e + 1], (GH, HID)),
             jnp.broadcast_to(valsB[:, e:e + 1], (GH, HID))], axis=1)
         for e in range(N_EDGES)], axis=0)


def _gnn_kernel(x_ref, ef_ref, ei_ref, *wrefs, out_ref):
    ws = [w[:] for w in wrefs]
    it = iter(ws)
    W_emb, b_emb = next(it), next(it)
    layers = [tuple(next(it) for _ in range(12)) for _ in range(N_LAYERS)]
    W_d1, b_d1, W_d2, b_d2 = next(it), next(it), next(it), next(it)

    ei = ei_ref[:]
    srcA, tgtA = ei[:, :20], ei[:, 20:40]
    srcB, tgtB = ei[:, 40:60], ei[:, 60:80]

    eft = ef_ref[:]
    eidxA = srcA * N + tgtA
    eidxB = srcB * N + tgtB
    z20 = jnp.zeros((GH, N_EDGES), jnp.float32)
    c0A, c1A, c0B, c1B = z20, z20, z20, z20
    for k in range(N * N):
        mA = (eidxA == k).astype(jnp.float32)
        mB = (eidxB == k).astype(jnp.float32)
        c0A = c0A + mA * eft[:, k:k + 1]
        c1A = c1A + mA * eft[:, 25 + k:26 + k]
        c0B = c0B + mB * eft[:, 50 + k:51 + k]
        c1B = c1B + mB * eft[:, 75 + k:76 + k]
    cp0 = _planes(c0A, c0B)
    cp1 = _planes(c1A, c1B)

    srcp = _planes(srcA.astype(jnp.float32), srcB.astype(jnp.float32))
    tgtp = _planes(tgtA.astype(jnp.float32), tgtB.astype(jnp.float32))
    srcm = [(srcp == n).astype(jnp.float32) for n in range(N)]
    tgtm = [(tgtp == n).astype(jnp.float32) for n in range(N)]

    x2d = _dot(x_ref[:].reshape(N * GH, 12), W_emb) + b_emb

    for li in range(N_LAYERS):
        eW1s, eW1t, w0, w1, eb1, eW2, eb2, nW1x, nW1a, nb1, nW2, nb2 = \
            layers[li]
        ys = _dot(x2d, eW1s).reshape(N, GH, 2 * HID)
        yt = _dot(x2d, eW1t).reshape(N, GH, 2 * HID)
        yx = _dot(x2d, nW1x)
        hs = cp0 * w0
        ht = cp1 * w1
        for n in range(N):
            hs = hs + srcm[n] * ys[n][None, :, :]
            ht = ht + tgtm[n] * yt[n][None, :, :]
        h = (hs + ht).reshape(N_EDGES * GH, 2 * HID) + eb1
        e = _silu(h)
        e = _silu(_dot(e, eW2) + eb2)
        e3 = e.reshape(N_EDGES, GH, 2 * HID)
        aggs = [jnp.sum(srcm[n] * e3, axis=0) for n in range(N)]
        agg2d = jnp.stack(aggs, axis=0).reshape(N * GH, 2 * HID)
        hn = _silu(yx + _dot(agg2d, nW1a) + nb1)
        x2d = _dot(hn, nW2) + nb2

    d = _silu(_dot(x2d, W_d1) + b_d1)
    out = _dot(d, W_d2) + b_d2
    out_ref[:] = out.reshape(N, GH, 6)


def _body(x_ref, ef_ref, ei_ref, *rest):
    _gnn_kernel(x_ref, ef_ref, ei_ref, *rest[:-1], out_ref=rest[-1])


@jax.jit
def kernel(node_features, edge_features, edge_idx, params):
    b = node_features.shape[0]
    bh = b // 2
    x_flat = node_features.transpose(0, 1, 3, 2).reshape(b, N, 3 * 2)
    x_pack = jnp.concatenate([x_flat[:bh], x_flat[bh:]], axis=2)
    x_in = x_pack.transpose(1, 0, 2)

    ef_flat = edge_features.reshape(b, N * N, 2)
    ef_half = jnp.concatenate([ef_flat[:, :, 0], ef_flat[:, :, 1]], axis=1)
    ef_in = jnp.concatenate([ef_half[:bh], ef_half[bh:]], axis=1)

    ei_flat = edge_idx.astype(jnp.int32).reshape(b, 2 * N_EDGES)
    ei_in = jnp.concatenate([ei_flat[:bh], ei_flat[bh:]], axis=1)

    weights = [_pair_w(params['W_emb']), _pair_b(params['b_emb'])]
    for i in range(N_LAYERS):
        p = params[f'layer_{i}']
        weights += [
            _pair_w(p['eW1'][:HID]), _pair_w(p['eW1'][HID:2 * HID]),
            jnp.tile(p['eW1'][2 * HID:2 * HID + 1], (1, 2)),
            jnp.tile(p['eW1'][2 * HID + 1:], (1, 2)),
            _pair_b(p['eb1']),
            _pair_w(p['eW2']), _pair_b(p['eb2']),
            _pair_w(p['nW1'][:HID]), _pair_w(p['nW1'][HID:]),
            _pair_b(p['nb1']),
            _pair_w(p['nW2']), _pair_b(p['nb2']),
        ]
    weights += [_pair_w(params['W_d1']), _pair_b(params['b_d1']),
                _pair_w(params['W_d2']), _pair_b(params['b_d2'])]

    grid = (bh // GH,)
    data_specs = [
        pl.BlockSpec((N, GH, 12), lambda i: (0, i, 0)),
        pl.BlockSpec((GH, 100), lambda i: (i, 0)),
        pl.BlockSpec((GH, 80), lambda i: (i, 0)),
    ]
    w_specs = [pl.BlockSpec(w.shape, functools.partial(lambda nd, i: (0,) * nd,
                                                       w.ndim))
               for w in weights]
    out = pl.pallas_call(
        _body,
        grid=grid,
        in_specs=data_specs + w_specs,
        out_specs=pl.BlockSpec((N, GH, 6), lambda i: (0, i, 0)),
        out_shape=jax.ShapeDtypeStruct((N, bh, 6), jnp.float32),
    )(x_in, ef_in, ei_in, *weights)
    out = out.transpose(1, 0, 2)
    out = jnp.concatenate([out[:, :, :3], out[:, :, 3:]], axis=0)
    return out


# speedup vs baseline: 4.0030x; 1.1849x over previous
import functools

import jax
import jax.numpy as jnp
from jax.experimental import pallas as pl

N = 5
N_EDGES = 20
HID = 64
N_LAYERS = 4
GH = 128


def _silu(x):
    return x * jax.nn.sigmoid(x)


def _dot(a, b):
    return jnp.dot(a, b, preferred_element_type=jnp.float32)


def _pair_w(W):
    z = jnp.zeros_like(W)
    return jnp.concatenate(
        [jnp.concatenate([W, z], axis=1), jnp.concatenate([z, W], axis=1)],
        axis=0)


def _pair_b(bias):
    return jnp.tile(bias.reshape(1, -1), (1, 2))


def _planes(valsA, valsB, m):
    return jnp.stack(
        [jnp.concatenate(
            [jnp.broadcast_to(valsA[:, e:e + 1], (GH, HID)),
             jnp.broadcast_to(valsB[:, e:e + 1], (GH, HID))], axis=1)
         for e in range(m)], axis=0)


def _gnn_kernel(x_ref, ef_ref, ei_ref, *wrefs, out_ref):
    ws = [w[:] for w in wrefs]
    it = iter(ws)
    W_emb, b_emb = next(it), next(it)
    layers = [tuple(next(it) for _ in range(12)) for _ in range(N_LAYERS)]
    W_d1, b_d1, W_d2, b_d2 = next(it), next(it), next(it), next(it)

    ei = ei_ref[:]
    srcA, tgtA = ei[:, :20], ei[:, 20:40]
    srcB, tgtB = ei[:, 40:60], ei[:, 60:80]

    eidxA = srcA * N + tgtA
    eidxB = srcB * N + tgtB
    cntA = jnp.concatenate(
        [jnp.sum((eidxA == k).astype(jnp.float32), axis=1, keepdims=True)
         for k in range(N * N)], axis=1)
    cntB = jnp.concatenate(
        [jnp.sum((eidxB == k).astype(jnp.float32), axis=1, keepdims=True)
         for k in range(N * N)], axis=1)
    cntp = _planes(cntA, cntB, N * N)

    eft = ef_ref[:]
    efs0 = _planes(eft[:, :25], eft[:, 50:75], N * N)
    efs1 = _planes(eft[:, 25:50], eft[:, 75:100], N * N)

    x2d = _dot(x_ref[:].reshape(N * GH, 12), W_emb) + b_emb

    for li in range(N_LAYERS):
        eW1s, eW1t, w0, w1, eb1, eW2, eb2, nW1x, nW1a, nb1, nW2, nb2 = \
            layers[li]
        ys = _dot(x2d, eW1s).reshape(N, GH, 2 * HID)
        yt = _dot(x2d, eW1t).reshape(N, GH, 2 * HID)
        yx = _dot(x2d, nW1x)
        hef = efs0 * w0 + efs1 * w1
        h = jnp.stack([ys[k // N] + yt[k % N] for k in range(N * N)],
                      axis=0) + hef
        h = h.reshape(N * N * GH, 2 * HID) + eb1
        e = _silu(h)
        e = _silu(_dot(e, eW2) + eb2)
        e3 = e.reshape(N * N, GH, 2 * HID)
        ew = cntp * e3
        aggs = [ew[r * N] + ew[r * N + 1] + ew[r * N + 2] + ew[r * N + 3]
                + ew[r * N + 4] for r in range(N)]
        agg2d = jnp.stack(aggs, axis=0).reshape(N * GH, 2 * HID)
        hn = _silu(yx + _dot(agg2d, nW1a) + nb1)
        x2d = _dot(hn, nW2) + nb2

    d = _silu(_dot(x2d, W_d1) + b_d1)
    out = _dot(d, W_d2) + b_d2
    out_ref[:] = out.reshape(N, GH, 6)


def _body(x_ref, ef_ref, ei_ref, *rest):
    _gnn_kernel(x_ref, ef_ref, ei_ref, *rest[:-1], out_ref=rest[-1])


@jax.jit
def kernel(node_features, edge_features, edge_idx, params):
    b = node_features.shape[0]
    bh = b // 2
    x_flat = node_features.transpose(0, 1, 3, 2).reshape(b, N, 3 * 2)
    x_pack = jnp.concatenate([x_flat[:bh], x_flat[bh:]], axis=2)
    x_in = x_pack.transpose(1, 0, 2)

    ef_flat = edge_features.reshape(b, N * N, 2)
    ef_half = jnp.concatenate([ef_flat[:, :, 0], ef_flat[:, :, 1]], axis=1)
    ef_in = jnp.concatenate([ef_half[:bh], ef_half[bh:]], axis=1)

    ei_flat = edge_idx.astype(jnp.int32).reshape(b, 2 * N_EDGES)
    ei_in = jnp.concatenate([ei_flat[:bh], ei_flat[bh:]], axis=1)

    weights = [_pair_w(params['W_emb']), _pair_b(params['b_emb'])]
    for i in range(N_LAYERS):
        p = params[f'layer_{i}']
        weights += [
            _pair_w(p['eW1'][:HID]), _pair_w(p['eW1'][HID:2 * HID]),
            jnp.tile(p['eW1'][2 * HID:2 * HID + 1], (1, 2)),
            jnp.tile(p['eW1'][2 * HID + 1:], (1, 2)),
            _pair_b(p['eb1']),
            _pair_w(p['eW2']), _pair_b(p['eb2']),
            _pair_w(p['nW1'][:HID]), _pair_w(p['nW1'][HID:]),
            _pair_b(p['nb1']),
            _pair_w(p['nW2']), _pair_b(p['nb2']),
        ]
    weights += [_pair_w(params['W_d1']), _pair_b(params['b_d1']),
                _pair_w(params['W_d2']), _pair_b(params['b_d2'])]

    grid = (bh // GH,)
    data_specs = [
        pl.BlockSpec((N, GH, 12), lambda i: (0, i, 0)),
        pl.BlockSpec((GH, 100), lambda i: (i, 0)),
        pl.BlockSpec((GH, 80), lambda i: (i, 0)),
    ]
    w_specs = [pl.BlockSpec(w.shape, functools.partial(lambda nd, i: (0,) * nd,
                                                       w.ndim))
               for w in weights]
    out = pl.pallas_call(
        _body,
        grid=grid,
        in_specs=data_specs + w_specs,
        out_specs=pl.BlockSpec((N, GH, 6), lambda i: (0, i, 0)),
        out_shape=jax.ShapeDtypeStruct((N, bh, 6), jnp.float32),
    )(x_in, ef_in, ei_in, *weights)
    out = out.transpose(1, 0, 2)
    out = jnp.concatenate([out[:, :, :3], out[:, :, 3:]], axis=0)
    return out


# speedup vs baseline: 4.6051x; 1.1504x over previous
import functools

import jax
import jax.numpy as jnp
from jax.experimental import pallas as pl

N = 5
N_EDGES = 20
HID = 64
N_LAYERS = 4
GH = 256


def _silu(x):
    return x * jax.nn.sigmoid(x)


def _dot(a, b):
    return jnp.dot(a, b, preferred_element_type=jnp.float32)


def _pair_w(W):
    z = jnp.zeros_like(W)
    return jnp.concatenate(
        [jnp.concatenate([W, z], axis=1), jnp.concatenate([z, W], axis=1)],
        axis=0)


def _pair_b(bias):
    return jnp.tile(bias.reshape(1, -1), (1, 2))


def _planes(valsA, valsB, m):
    return jnp.stack(
        [jnp.concatenate(
            [jnp.broadcast_to(valsA[:, e:e + 1], (GH, HID)),
             jnp.broadcast_to(valsB[:, e:e + 1], (GH, HID))], axis=1)
         for e in range(m)], axis=0)


def _gnn_kernel(x_ref, ef_ref, ei_ref, *wrefs, out_ref):
    ws = [w[:] for w in wrefs]
    it = iter(ws)
    W_emb, b_emb = next(it), next(it)
    layers = [tuple(next(it) for _ in range(12)) for _ in range(N_LAYERS)]
    W_d1, b_d1, W_d2, b_d2 = next(it), next(it), next(it), next(it)

    ei = ei_ref[:]
    srcA, tgtA = ei[:, :20], ei[:, 20:40]
    srcB, tgtB = ei[:, 40:60], ei[:, 60:80]

    eidxA = srcA * N + tgtA
    eidxB = srcB * N + tgtB
    cntA = jnp.concatenate(
        [jnp.sum((eidxA == k).astype(jnp.float32), axis=1, keepdims=True)
         for k in range(N * N)], axis=1)
    cntB = jnp.concatenate(
        [jnp.sum((eidxB == k).astype(jnp.float32), axis=1, keepdims=True)
         for k in range(N * N)], axis=1)
    cntp = _planes(cntA, cntB, N * N)

    eft = ef_ref[:]
    efs0 = _planes(eft[:, :25], eft[:, 50:75], N * N)
    efs1 = _planes(eft[:, 25:50], eft[:, 75:100], N * N)

    x2d = _dot(x_ref[:].reshape(N * GH, 12), W_emb) + b_emb

    for li in range(N_LAYERS):
        eW1s, eW1t, w0, w1, eb1, eW2, eb2, nW1x, nW1a, nb1, nW2, nb2 = \
            layers[li]
        ys = _dot(x2d, eW1s).reshape(N, GH, 2 * HID)
        yt = _dot(x2d, eW1t).reshape(N, GH, 2 * HID)
        yx = _dot(x2d, nW1x)
        hef = efs0 * w0 + efs1 * w1
        h = jnp.stack([ys[k // N] + yt[k % N] for k in range(N * N)],
                      axis=0) + hef
        h = h.reshape(N * N * GH, 2 * HID) + eb1
        e = _silu(h)
        e = _silu(_dot(e, eW2) + eb2)
        e3 = e.reshape(N * N, GH, 2 * HID)
        ew = cntp * e3
        aggs = [ew[r * N] + ew[r * N + 1] + ew[r * N + 2] + ew[r * N + 3]
                + ew[r * N + 4] for r in range(N)]
        agg2d = jnp.stack(aggs, axis=0).reshape(N * GH, 2 * HID)
        hn = _silu(yx + _dot(agg2d, nW1a) + nb1)
        x2d = _dot(hn, nW2) + nb2

    d = _silu(_dot(x2d, W_d1) + b_d1)
    out = _dot(d, W_d2) + b_d2
    out_ref[:] = out.reshape(N, GH, 6)


def _body(x_ref, ef_ref, ei_ref, *rest):
    _gnn_kernel(x_ref, ef_ref, ei_ref, *rest[:-1], out_ref=rest[-1])


@jax.jit
def kernel(node_features, edge_features, edge_idx, params):
    b = node_features.shape[0]
    bh = b // 2
    x_flat = node_features.transpose(0, 1, 3, 2).reshape(b, N, 3 * 2)
    x_pack = jnp.concatenate([x_flat[:bh], x_flat[bh:]], axis=2)
    x_in = x_pack.transpose(1, 0, 2)

    ef_flat = edge_features.reshape(b, N * N, 2)
    ef_half = jnp.concatenate([ef_flat[:, :, 0], ef_flat[:, :, 1]], axis=1)
    ef_in = jnp.concatenate([ef_half[:bh], ef_half[bh:]], axis=1)

    ei_flat = edge_idx.astype(jnp.int32).reshape(b, 2 * N_EDGES)
    ei_in = jnp.concatenate([ei_flat[:bh], ei_flat[bh:]], axis=1)

    weights = [_pair_w(params['W_emb']), _pair_b(params['b_emb'])]
    for i in range(N_LAYERS):
        p = params[f'layer_{i}']
        weights += [
            _pair_w(p['eW1'][:HID]), _pair_w(p['eW1'][HID:2 * HID]),
            jnp.tile(p['eW1'][2 * HID:2 * HID + 1], (1, 2)),
            jnp.tile(p['eW1'][2 * HID + 1:], (1, 2)),
            _pair_b(p['eb1']),
            _pair_w(p['eW2']), _pair_b(p['eb2']),
            _pair_w(p['nW1'][:HID]), _pair_w(p['nW1'][HID:]),
            _pair_b(p['nb1']),
            _pair_w(p['nW2']), _pair_b(p['nb2']),
        ]
    weights += [_pair_w(params['W_d1']), _pair_b(params['b_d1']),
                _pair_w(params['W_d2']), _pair_b(params['b_d2'])]

    grid = (bh // GH,)
    data_specs = [
        pl.BlockSpec((N, GH, 12), lambda i: (0, i, 0)),
        pl.BlockSpec((GH, 100), lambda i: (i, 0)),
        pl.BlockSpec((GH, 80), lambda i: (i, 0)),
    ]
    w_specs = [pl.BlockSpec(w.shape, functools.partial(lambda nd, i: (0,) * nd,
                                                       w.ndim))
               for w in weights]
    out = pl.pallas_call(
        _body,
        grid=grid,
        in_specs=data_specs + w_specs,
        out_specs=pl.BlockSpec((N, GH, 6), lambda i: (0, i, 0)),
        out_shape=jax.ShapeDtypeStruct((N, bh, 6), jnp.float32),
    )(x_in, ef_in, ei_in, *weights)
    out = out.transpose(1, 0, 2)
    out = jnp.concatenate([out[:, :, :3], out[:, :, 3:]], axis=0)
    return out


# speedup vs baseline: 4.8132x; 1.0452x over previous
import functools

import jax
import jax.numpy as jnp
from jax.experimental import pallas as pl

N = 5
N_EDGES = 20
HID = 64
N_LAYERS = 4
GH = 512


def _silu(x):
    return x * jax.nn.sigmoid(x)


def _dot(a, b):
    return jnp.dot(a, b, preferred_element_type=jnp.float32)


def _pair_w(W):
    z = jnp.zeros_like(W)
    return jnp.concatenate(
        [jnp.concatenate([W, z], axis=1), jnp.concatenate([z, W], axis=1)],
        axis=0)


def _pair_b(bias):
    return jnp.tile(bias.reshape(1, -1), (1, 2))


def _planes(valsA, valsB, m):
    return jnp.stack(
        [jnp.concatenate(
            [jnp.broadcast_to(valsA[:, e:e + 1], (GH, HID)),
             jnp.broadcast_to(valsB[:, e:e + 1], (GH, HID))], axis=1)
         for e in range(m)], axis=0)


def _gnn_kernel(x_ref, ef_ref, ei_ref, *wrefs, out_ref):
    ws = [w[:] for w in wrefs]
    it = iter(ws)
    W_emb, b_emb = next(it), next(it)
    layers = [tuple(next(it) for _ in range(12)) for _ in range(N_LAYERS)]
    W_d1, b_d1, W_d2, b_d2 = next(it), next(it), next(it), next(it)

    ei = ei_ref[:]
    srcA, tgtA = ei[:, :20], ei[:, 20:40]
    srcB, tgtB = ei[:, 40:60], ei[:, 60:80]

    eidxA = srcA * N + tgtA
    eidxB = srcB * N + tgtB
    cntA = jnp.concatenate(
        [jnp.sum((eidxA == k).astype(jnp.float32), axis=1, keepdims=True)
         for k in range(N * N)], axis=1)
    cntB = jnp.concatenate(
        [jnp.sum((eidxB == k).astype(jnp.float32), axis=1, keepdims=True)
         for k in range(N * N)], axis=1)
    cntp = _planes(cntA, cntB, N * N)

    eft = ef_ref[:]
    efs0 = _planes(eft[:, :25], eft[:, 50:75], N * N)
    efs1 = _planes(eft[:, 25:50], eft[:, 75:100], N * N)

    x2d = _dot(x_ref[:].reshape(N * GH, 12), W_emb) + b_emb

    for li in range(N_LAYERS):
        eW1s, eW1t, w0, w1, eb1, eW2, eb2, nW1x, nW1a, nb1, nW2, nb2 = \
            layers[li]
        ys = _dot(x2d, eW1s).reshape(N, GH, 2 * HID)
        yt = _dot(x2d, eW1t).reshape(N, GH, 2 * HID)
        yx = _dot(x2d, nW1x)
        hef = efs0 * w0 + efs1 * w1
        h = jnp.stack([ys[k // N] + yt[k % N] for k in range(N * N)],
                      axis=0) + hef
        h = h.reshape(N * N * GH, 2 * HID) + eb1
        e = _silu(h)
        e = _silu(_dot(e, eW2) + eb2)
        e3 = e.reshape(N * N, GH, 2 * HID)
        ew = cntp * e3
        aggs = [ew[r * N] + ew[r * N + 1] + ew[r * N + 2] + ew[r * N + 3]
                + ew[r * N + 4] for r in range(N)]
        agg2d = jnp.stack(aggs, axis=0).reshape(N * GH, 2 * HID)
        hn = _silu(yx + _dot(agg2d, nW1a) + nb1)
        x2d = _dot(hn, nW2) + nb2

    d = _silu(_dot(x2d, W_d1) + b_d1)
    out = _dot(d, W_d2) + b_d2
    out_ref[:] = out.reshape(N, GH, 6)


def _body(x_ref, ef_ref, ei_ref, *rest):
    _gnn_kernel(x_ref, ef_ref, ei_ref, *rest[:-1], out_ref=rest[-1])


@jax.jit
def kernel(node_features, edge_features, edge_idx, params):
    b = node_features.shape[0]
    bh = b // 2
    x_flat = node_features.transpose(0, 1, 3, 2).reshape(b, N, 3 * 2)
    x_pack = jnp.concatenate([x_flat[:bh], x_flat[bh:]], axis=2)
    x_in = x_pack.transpose(1, 0, 2)

    ef_flat = edge_features.reshape(b, N * N, 2)
    ef_half = jnp.concatenate([ef_flat[:, :, 0], ef_flat[:, :, 1]], axis=1)
    ef_in = jnp.concatenate([ef_half[:bh], ef_half[bh:]], axis=1)

    ei_flat = edge_idx.astype(jnp.int32).reshape(b, 2 * N_EDGES)
    ei_in = jnp.concatenate([ei_flat[:bh], ei_flat[bh:]], axis=1)

    weights = [_pair_w(params['W_emb']), _pair_b(params['b_emb'])]
    for i in range(N_LAYERS):
        p = params[f'layer_{i}']
        weights += [
            _pair_w(p['eW1'][:HID]), _pair_w(p['eW1'][HID:2 * HID]),
            jnp.tile(p['eW1'][2 * HID:2 * HID + 1], (1, 2)),
            jnp.tile(p['eW1'][2 * HID + 1:], (1, 2)),
            _pair_b(p['eb1']),
            _pair_w(p['eW2']), _pair_b(p['eb2']),
            _pair_w(p['nW1'][:HID]), _pair_w(p['nW1'][HID:]),
            _pair_b(p['nb1']),
            _pair_w(p['nW2']), _pair_b(p['nb2']),
        ]
    weights += [_pair_w(params['W_d1']), _pair_b(params['b_d1']),
                _pair_w(params['W_d2']), _pair_b(params['b_d2'])]

    grid = (bh // GH,)
    data_specs = [
        pl.BlockSpec((N, GH, 12), lambda i: (0, i, 0)),
        pl.BlockSpec((GH, 100), lambda i: (i, 0)),
        pl.BlockSpec((GH, 80), lambda i: (i, 0)),
    ]
    w_specs = [pl.BlockSpec(w.shape, functools.partial(lambda nd, i: (0,) * nd,
                                                       w.ndim))
               for w in weights]
    out = pl.pallas_call(
        _body,
        grid=grid,
        in_specs=data_specs + w_specs,
        out_specs=pl.BlockSpec((N, GH, 6), lambda i: (0, i, 0)),
        out_shape=jax.ShapeDtypeStruct((N, bh, 6), jnp.float32),
    )(x_in, ef_in, ei_in, *weights)
    out = out.transpose(1, 0, 2)
    out = jnp.concatenate([out[:, :, :3], out[:, :, 3:]], axis=0)
    return out


# speedup vs baseline: 5.3492x; 1.1114x over previous
import functools

import jax
import jax.numpy as jnp
from jax.experimental import pallas as pl

N = 5
N_EDGES = 20
HID = 64
N_LAYERS = 4
GH = 512


def _silu(x):
    return x * jax.nn.sigmoid(x)


def _dot(a, b):
    return jnp.dot(a, b, preferred_element_type=jnp.float32)


def _pair_w(W):
    z = jnp.zeros_like(W)
    return jnp.concatenate(
        [jnp.concatenate([W, z], axis=1), jnp.concatenate([z, W], axis=1)],
        axis=0)


def _pair_b(bias):
    return jnp.tile(bias.reshape(1, -1), (1, 2))


def _planes(valsA, valsB, m):
    return jnp.stack(
        [jnp.concatenate(
            [jnp.broadcast_to(valsA[:, e:e + 1], (GH, HID)),
             jnp.broadcast_to(valsB[:, e:e + 1], (GH, HID))], axis=1)
         for e in range(m)], axis=0)


def _gnn_kernel(x_ref, ef_ref, ei_ref, *wrefs, out_ref):
    ws = [w[:] for w in wrefs]
    it = iter(ws)
    W_emb, b_emb = next(it), next(it)
    layers = [tuple(next(it) for _ in range(11)) for _ in range(N_LAYERS)]
    W_d1, b_d1, W_d2, b_d2 = next(it), next(it), next(it), next(it)

    ei = ei_ref[:]
    srcA, tgtA = ei[:, :20], ei[:, 20:40]
    srcB, tgtB = ei[:, 40:60], ei[:, 60:80]

    eidxA = srcA * N + tgtA
    eidxB = srcB * N + tgtB
    cntA = jnp.concatenate(
        [jnp.sum((eidxA == k).astype(jnp.float32), axis=1, keepdims=True)
         for k in range(N * N)], axis=1)
    cntB = jnp.concatenate(
        [jnp.sum((eidxB == k).astype(jnp.float32), axis=1, keepdims=True)
         for k in range(N * N)], axis=1)
    cntp = _planes(cntA, cntB, N * N)

    E4 = ef_ref[:].reshape(N * N * GH, 4)

    x2d = _dot(x_ref[:].reshape(N * GH, 12), W_emb) + b_emb

    for li in range(N_LAYERS):
        eW1s, eW1t, W4, eb1, eW2, eb2, nW1x, nW1a, nb1, nW2, nb2 = \
            layers[li]
        ys = _dot(x2d, eW1s).reshape(N, GH, 2 * HID)
        yt = _dot(x2d, eW1t).reshape(N, GH, 2 * HID)
        yx = _dot(x2d, nW1x)
        hef = _dot(E4, W4) + eb1
        h = jnp.stack([ys[k // N] + yt[k % N] for k in range(N * N)],
                      axis=0)
        h = h.reshape(N * N * GH, 2 * HID) + hef
        e = _silu(h)
        e = _silu(_dot(e, eW2) + eb2)
        e3 = e.reshape(N * N, GH, 2 * HID)
        ew = cntp * e3
        aggs = [ew[r * N] + ew[r * N + 1] + ew[r * N + 2] + ew[r * N + 3]
                + ew[r * N + 4] for r in range(N)]
        agg2d = jnp.stack(aggs, axis=0).reshape(N * GH, 2 * HID)
        hn = _silu(yx + _dot(agg2d, nW1a) + nb1)
        x2d = _dot(hn, nW2) + nb2

    d = _silu(_dot(x2d, W_d1) + b_d1)
    out = _dot(d, W_d2) + b_d2
    out_ref[:] = out.reshape(N, GH, 6)


def _body(x_ref, ef_ref, ei_ref, *rest):
    _gnn_kernel(x_ref, ef_ref, ei_ref, *rest[:-1], out_ref=rest[-1])


@jax.jit
def kernel(node_features, edge_features, edge_idx, params):
    b = node_features.shape[0]
    bh = b // 2
    x_flat = node_features.transpose(0, 1, 3, 2).reshape(b, N, 3 * 2)
    x_pack = jnp.concatenate([x_flat[:bh], x_flat[bh:]], axis=2)
    x_in = x_pack.transpose(1, 0, 2)

    ef_flat = edge_features.reshape(b, N * N, 2)
    ef_pack = jnp.concatenate([ef_flat[:bh], ef_flat[bh:]], axis=2)
    ef_in = ef_pack.transpose(1, 0, 2)

    ei_flat = edge_idx.astype(jnp.int32).reshape(b, 2 * N_EDGES)
    ei_in = jnp.concatenate([ei_flat[:bh], ei_flat[bh:]], axis=1)

    weights = [_pair_w(params['W_emb']), _pair_b(params['b_emb'])]
    for i in range(N_LAYERS):
        p = params[f'layer_{i}']
        weights += [
            _pair_w(p['eW1'][:HID]), _pair_w(p['eW1'][HID:2 * HID]),
            _pair_w(p['eW1'][2 * HID:]),
            _pair_b(p['eb1']),
            _pair_w(p['eW2']), _pair_b(p['eb2']),
            _pair_w(p['nW1'][:HID]), _pair_w(p['nW1'][HID:]),
            _pair_b(p['nb1']),
            _pair_w(p['nW2']), _pair_b(p['nb2']),
        ]
    weights += [_pair_w(params['W_d1']), _pair_b(params['b_d1']),
                _pair_w(params['W_d2']), _pair_b(params['b_d2'])]

    grid = (bh // GH,)
    data_specs = [
        pl.BlockSpec((N, GH, 12), lambda i: (0, i, 0)),
        pl.BlockSpec((N * N, GH, 4), lambda i: (0, i, 0)),
        pl.BlockSpec((GH, 80), lambda i: (i, 0)),
    ]
    w_specs = [pl.BlockSpec(w.shape, functools.partial(lambda nd, i: (0,) * nd,
                                                       w.ndim))
               for w in weights]
    out = pl.pallas_call(
        _body,
        grid=grid,
        in_specs=data_specs + w_specs,
        out_specs=pl.BlockSpec((N, GH, 6), lambda i: (0, i, 0)),
        out_shape=jax.ShapeDtypeStruct((N, bh, 6), jnp.float32),
    )(x_in, ef_in, ei_in, *weights)
    out = out.transpose(1, 0, 2)
    out = jnp.concatenate([out[:, :, :3], out[:, :, 3:]], axis=0)
    return out


# speedup vs baseline: 5.4578x; 1.0203x over previous
import functools

import jax
import jax.numpy as jnp
from jax.experimental import pallas as pl

N = 5
N_EDGES = 20
HID = 64
N_LAYERS = 4
GH = 512


def _silu(x):
    return x * (0.5 * jnp.tanh(0.5 * x) + 0.5)


def _dot(a, b):
    return jnp.dot(a, b, preferred_element_type=jnp.float32)


def _pair_w(W):
    z = jnp.zeros_like(W)
    return jnp.concatenate(
        [jnp.concatenate([W, z], axis=1), jnp.concatenate([z, W], axis=1)],
        axis=0)


def _pair_b(bias):
    return jnp.tile(bias.reshape(1, -1), (1, 2))


def _planes(valsA, valsB, m):
    return jnp.stack(
        [jnp.concatenate(
            [jnp.broadcast_to(valsA[:, e:e + 1], (GH, HID)),
             jnp.broadcast_to(valsB[:, e:e + 1], (GH, HID))], axis=1)
         for e in range(m)], axis=0)


def _gnn_kernel(x_ref, ef_ref, ei_ref, *wrefs, out_ref):
    ws = [w[:] for w in wrefs]
    it = iter(ws)
    W_emb, b_emb = next(it), next(it)
    layers = [tuple(next(it) for _ in range(11)) for _ in range(N_LAYERS)]
    W_d1, b_d1, W_d2, b_d2 = next(it), next(it), next(it), next(it)

    ei = ei_ref[:]
    srcA, tgtA = ei[:, :20], ei[:, 20:40]
    srcB, tgtB = ei[:, 40:60], ei[:, 60:80]

    eidxA = srcA * N + tgtA
    eidxB = srcB * N + tgtB
    cntA = jnp.concatenate(
        [jnp.sum((eidxA == k).astype(jnp.float32), axis=1, keepdims=True)
         for k in range(N * N)], axis=1)
    cntB = jnp.concatenate(
        [jnp.sum((eidxB == k).astype(jnp.float32), axis=1, keepdims=True)
         for k in range(N * N)], axis=1)
    cntp = _planes(cntA, cntB, N * N)

    E4 = ef_ref[:].reshape(N * N * GH, 4)

    x2d = _dot(x_ref[:].reshape(N * GH, 12), W_emb) + b_emb

    for li in range(N_LAYERS):
        eW1s, eW1t, W4, eb1, eW2, eb2, nW1x, nW1a, nb1, nW2, nb2 = \
            layers[li]
        ys = _dot(x2d, eW1s).reshape(N, GH, 2 * HID)
        yt = _dot(x2d, eW1t).reshape(N, GH, 2 * HID)
        yx = _dot(x2d, nW1x)
        hef = _dot(E4, W4) + eb1
        h = jnp.stack([ys[k // N] + yt[k % N] for k in range(N * N)],
                      axis=0)
        h = h.reshape(N * N * GH, 2 * HID) + hef
        e = _silu(h)
        e = _silu(_dot(e, eW2) + eb2)
        e3 = e.reshape(N * N, GH, 2 * HID)
        ew = cntp * e3
        aggs = [ew[r * N] + ew[r * N + 1] + ew[r * N + 2] + ew[r * N + 3]
                + ew[r * N + 4] for r in range(N)]
        agg2d = jnp.stack(aggs, axis=0).reshape(N * GH, 2 * HID)
        hn = _silu(yx + _dot(agg2d, nW1a) + nb1)
        x2d = _dot(hn, nW2) + nb2

    d = _silu(_dot(x2d, W_d1) + b_d1)
    out = _dot(d, W_d2) + b_d2
    out_ref[:] = out.reshape(N, GH, 6)


def _body(x_ref, ef_ref, ei_ref, *rest):
    _gnn_kernel(x_ref, ef_ref, ei_ref, *rest[:-1], out_ref=rest[-1])


@jax.jit
def kernel(node_features, edge_features, edge_idx, params):
    b = node_features.shape[0]
    bh = b // 2
    x_flat = node_features.transpose(0, 1, 3, 2).reshape(b, N, 3 * 2)
    x_pack = jnp.concatenate([x_flat[:bh], x_flat[bh:]], axis=2)
    x_in = x_pack.transpose(1, 0, 2)

    ef_flat = edge_features.reshape(b, N * N, 2)
    ef_pack = jnp.concatenate([ef_flat[:bh], ef_flat[bh:]], axis=2)
    ef_in = ef_pack.transpose(1, 0, 2)

    ei_flat = edge_idx.astype(jnp.int32).reshape(b, 2 * N_EDGES)
    ei_in = jnp.concatenate([ei_flat[:bh], ei_flat[bh:]], axis=1)

    weights = [_pair_w(params['W_emb']), _pair_b(params['b_emb'])]
    for i in range(N_LAYERS):
        p = params[f'layer_{i}']
        weights += [
            _pair_w(p['eW1'][:HID]), _pair_w(p['eW1'][HID:2 * HID]),
            _pair_w(p['eW1'][2 * HID:]),
            _pair_b(p['eb1']),
            _pair_w(p['eW2']), _pair_b(p['eb2']),
            _pair_w(p['nW1'][:HID]), _pair_w(p['nW1'][HID:]),
            _pair_b(p['nb1']),
            _pair_w(p['nW2']), _pair_b(p['nb2']),
        ]
    weights += [_pair_w(params['W_d1']), _pair_b(params['b_d1']),
                _pair_w(params['W_d2']), _pair_b(params['b_d2'])]

    grid = (bh // GH,)
    data_specs = [
        pl.BlockSpec((N, GH, 12), lambda i: (0, i, 0)),
        pl.BlockSpec((N * N, GH, 4), lambda i: (0, i, 0)),
        pl.BlockSpec((GH, 80), lambda i: (i, 0)),
    ]
    w_specs = [pl.BlockSpec(w.shape, functools.partial(lambda nd, i: (0,) * nd,
                                                       w.ndim))
               for w in weights]
    out = pl.pallas_call(
        _body,
        grid=grid,
        in_specs=data_specs + w_specs,
        out_specs=pl.BlockSpec((N, GH, 6), lambda i: (0, i, 0)),
        out_shape=jax.ShapeDtypeStruct((N, bh, 6), jnp.float32),
    )(x_in, ef_in, ei_in, *weights)
    out = out.transpose(1, 0, 2)
    out = jnp.concatenate([out[:, :, :3], out[:, :, 3:]], axis=0)
    return out
